# Initial kernel scaffold; baseline (speedup 1.0000x reference)
import jax
import jax.numpy as jnp
from jax.experimental import pallas as pl


def kernel(x, table, W, b, bn_gamma, bn_beta, bn_mean, bn_var, ln_gamma, ln_beta):
    raise NotImplementedError("write your pallas kernel here")


# speedup vs baseline: 7.2505x; 7.2505x over previous
import functools

import jax
import jax.numpy as jnp
import numpy as np
from jax import lax
from jax.experimental import pallas as pl
from jax.experimental.pallas import tpu as pltpu
from jax.experimental.pallas import tpu_sc as plsc

B, L, V, F = 16384, 20, 100000, 128
NC, NS = 2, 16
NW = NC * NS
ROWS_PER_W = B // NW
IDX_PER_W = ROWS_PER_W * L
SLAB = 128
NSLAB = IDX_PER_W // SLAB

_SEG_NP = (
    np.arange(NS, dtype=np.int32)[:, None] * ROWS_PER_W
    + (np.arange(NSLAB * SLAB, dtype=np.int32) // L)[None, :]
).reshape(NS, NSLAB, SLAB)

_EPS = 1e-3
_VEC = 16


def _sc_pool_body(x_r, table, seg, out, idx_v, seg_v, rows0, rows1, acc_sh,
                  sem0, sem1):
    cid = lax.axis_index("c")
    sid = lax.axis_index("s")
    wid = sid * NC + cid

    pltpu.sync_copy(x_r.at[wid], idx_v)
    pltpu.sync_copy(seg.at[sid], seg_v)

    zero = jnp.zeros((_VEC,), jnp.float32)

    @pl.loop(0, SLAB)
    def _zero(r):
        for c in range(F // _VEC):
            rows0[r, pl.ds(c * _VEC, _VEC)] = zero

    for q in range(ROWS_PER_W // SLAB):
        pltpu.sync_copy(rows0, acc_sh.at[pl.ds(sid * ROWS_PER_W + q * SLAB, SLAB)])

    @pl.loop(0, NSLAB, step=2)
    def _slabs(j0):
        j1 = j0 + 1
        c0 = pltpu.async_copy(table.at[idx_v.at[j0]], rows0, sem0)
        c1 = pltpu.async_copy(table.at[idx_v.at[j1]], rows1, sem1)
        c0.wait()
        pltpu.sync_copy(rows0, acc_sh.at[seg_v.at[j0]], add=True)
        c1.wait()
        pltpu.sync_copy(rows1, acc_sh.at[seg_v.at[j1]], add=True)

    pltpu.sync_copy(acc_sh.at[pl.ds(sid * ROWS_PER_W, ROWS_PER_W)],
                    out.at[pl.ds(wid * ROWS_PER_W, ROWS_PER_W)])


_sc_pool = functools.partial(
    pl.kernel,
    out_type=jax.ShapeDtypeStruct((B, F), jnp.float32),
    mesh=plsc.VectorSubcoreMesh(core_axis_name="c", subcore_axis_name="s"),
    scratch_types=[
        pltpu.VMEM((NSLAB, SLAB), jnp.int32),
        pltpu.VMEM((NSLAB, SLAB), jnp.int32),
        pltpu.VMEM((SLAB, F), jnp.float32),
        pltpu.VMEM((SLAB, F), jnp.float32),
        pltpu.VMEM_SHARED((NS * ROWS_PER_W, F), jnp.float32),
        pltpu.SemaphoreType.DMA,
        pltpu.SemaphoreType.DMA,
    ],
)(_sc_pool_body)


def _tc_body(pooled_ref, W_ref, b_ref, bng_ref, bnb_ref, bnm_ref, bnv_ref,
             lng_ref, lnb_ref, out_ref):
    s = bng_ref[...] * lax.rsqrt(bnv_ref[...] + _EPS)
    t = bnb_ref[...] - bnm_ref[...] * s
    z = jnp.dot(pooled_ref[...], W_ref[...],
                preferred_element_type=jnp.float32)
    z = z * (s * (1.0 / L)) + (b_ref[...] * s + t)
    mu = jnp.mean(z, axis=-1, keepdims=True)
    var = jnp.mean(jnp.square(z - mu), axis=-1, keepdims=True)
    out_ref[...] = (z - mu) * lax.rsqrt(var + _EPS) * lng_ref[...] + lnb_ref[...]


_TC_BLK = 1024


def _tc_head(pooled, W, b, bn_gamma, bn_beta, bn_mean, bn_var, ln_gamma, ln_beta):
    vec_spec = pl.BlockSpec((1, F), lambda i: (0, 0))
    return pl.pallas_call(
        _tc_body,
        grid=(B // _TC_BLK,),
        in_specs=[
            pl.BlockSpec((_TC_BLK, F), lambda i: (i, 0)),
            pl.BlockSpec((F, F), lambda i: (0, 0)),
            vec_spec, vec_spec, vec_spec, vec_spec, vec_spec, vec_spec, vec_spec,
        ],
        out_specs=pl.BlockSpec((_TC_BLK, F), lambda i: (i, 0)),
        out_shape=jax.ShapeDtypeStruct((B, F), jnp.float32),
    )(pooled, W, b.reshape(1, F), bn_gamma.reshape(1, F), bn_beta.reshape(1, F),
      bn_mean.reshape(1, F), bn_var.reshape(1, F), ln_gamma.reshape(1, F),
      ln_beta.reshape(1, F))


def kernel(x, table, W, b, bn_gamma, bn_beta, bn_mean, bn_var, ln_gamma, ln_beta):
    x_r = x.astype(jnp.int32).reshape(NW, NSLAB, SLAB)
    seg = jnp.asarray(_SEG_NP)
    pooled = _sc_pool(x_r, table, seg)
    return _tc_head(pooled, W, b, bn_gamma, bn_beta, bn_mean, bn_var,
                    ln_gamma, ln_beta)


# speedup vs baseline: 8.9116x; 1.2291x over previous
import functools

import jax
import jax.numpy as jnp
import numpy as np
from jax import lax
from jax.experimental import pallas as pl
from jax.experimental.pallas import tpu as pltpu
from jax.experimental.pallas import tpu_sc as plsc

B, L, V, F = 16384, 20, 100000, 128
NC, NS = 2, 16
NW = NC * NS
ROWS_PER_W = B // NW
IDX_PER_W = ROWS_PER_W * L
SLAB = 128
NSLAB = IDX_PER_W // SLAB

NPHASE = ROWS_PER_W // SLAB
PH_SLABS = NSLAB // NPHASE
_SEG_NP = (
    np.arange(NS, dtype=np.int32)[:, None] * SLAB
    + (np.arange(PH_SLABS * SLAB, dtype=np.int32) // L)[None, :]
).reshape(NS, PH_SLABS, SLAB)

_EPS = 1e-3
_VEC = 16


_NBUF = 4


def _sc_pool_body(x_r, table, seg, out, idx_v, seg_v, zbuf, rows, acc_sh, *sems):
    cid = lax.axis_index("c")
    sid = lax.axis_index("s")
    wid = sid * NC + cid

    pltpu.sync_copy(x_r.at[wid], idx_v)
    pltpu.sync_copy(seg.at[sid], seg_v)

    zero = jnp.zeros((_VEC,), jnp.float32)

    @pl.loop(0, SLAB)
    def _zero(r):
        for c in range(F // _VEC):
            zbuf[r, pl.ds(c * _VEC, _VEC)] = zero

    acc_base = sid * SLAB
    for p in range(NPHASE):
        pltpu.sync_copy(zbuf, acc_sh.at[pl.ds(acc_base, SLAB)])
        for b in range(_NBUF):
            pltpu.async_copy(table.at[idx_v.at[p * PH_SLABS + b]],
                             rows.at[b], sems[b])

        @pl.loop(0, PH_SLABS, step=_NBUF)
        def _slabs(j):
            for b in range(_NBUF):
                pltpu.make_async_copy(table.at[pl.ds(0, SLAB)], rows.at[b],
                                      sems[b]).wait()
                pltpu.sync_copy(rows.at[b], acc_sh.at[seg_v.at[j + b]],
                                add=True)
                nxt = j + _NBUF + b

                @pl.when(nxt < PH_SLABS)
                def _():
                    pltpu.async_copy(table.at[idx_v.at[p * PH_SLABS + nxt]],
                                     rows.at[b], sems[b])

        pltpu.sync_copy(acc_sh.at[pl.ds(acc_base, SLAB)],
                        out.at[pl.ds(wid * ROWS_PER_W + p * SLAB, SLAB)])


_sc_pool = functools.partial(
    pl.kernel,
    out_type=jax.ShapeDtypeStruct((B, F), jnp.float32),
    mesh=plsc.VectorSubcoreMesh(core_axis_name="c", subcore_axis_name="s"),
    scratch_types=[
        pltpu.VMEM((NSLAB, SLAB), jnp.int32),
        pltpu.VMEM((PH_SLABS, SLAB), jnp.int32),
        pltpu.VMEM((SLAB, F), jnp.float32),
        pltpu.VMEM((_NBUF, SLAB, F), jnp.float32),
        pltpu.VMEM_SHARED((NS * SLAB, F), jnp.float32),
    ] + [pltpu.SemaphoreType.DMA] * _NBUF,
)(_sc_pool_body)


def _tc_body(pooled_ref, W_ref, b_ref, bng_ref, bnb_ref, bnm_ref, bnv_ref,
             lng_ref, lnb_ref, out_ref):
    s = bng_ref[...] * lax.rsqrt(bnv_ref[...] + _EPS)
    t = bnb_ref[...] - bnm_ref[...] * s
    z = jnp.dot(pooled_ref[...], W_ref[...],
                preferred_element_type=jnp.float32)
    z = z * (s * (1.0 / L)) + (b_ref[...] * s + t)
    mu = jnp.mean(z, axis=-1, keepdims=True)
    var = jnp.mean(jnp.square(z - mu), axis=-1, keepdims=True)
    out_ref[...] = (z - mu) * lax.rsqrt(var + _EPS) * lng_ref[...] + lnb_ref[...]


_TC_BLK = 1024


def _tc_head(pooled, W, b, bn_gamma, bn_beta, bn_mean, bn_var, ln_gamma, ln_beta):
    vec_spec = pl.BlockSpec((1, F), lambda i: (0, 0))
    return pl.pallas_call(
        _tc_body,
        grid=(B // _TC_BLK,),
        in_specs=[
            pl.BlockSpec((_TC_BLK, F), lambda i: (i, 0)),
            pl.BlockSpec((F, F), lambda i: (0, 0)),
            vec_spec, vec_spec, vec_spec, vec_spec, vec_spec, vec_spec, vec_spec,
        ],
        out_specs=pl.BlockSpec((_TC_BLK, F), lambda i: (i, 0)),
        out_shape=jax.ShapeDtypeStruct((B, F), jnp.float32),
    )(pooled, W, b.reshape(1, F), bn_gamma.reshape(1, F), bn_beta.reshape(1, F),
      bn_mean.reshape(1, F), bn_var.reshape(1, F), ln_gamma.reshape(1, F),
      ln_beta.reshape(1, F))


def kernel(x, table, W, b, bn_gamma, bn_beta, bn_mean, bn_var, ln_gamma, ln_beta):
    x_r = x.astype(jnp.int32).reshape(NW, NSLAB, SLAB)
    seg = jnp.asarray(_SEG_NP)
    pooled = _sc_pool(x_r, table, seg)
    return _tc_head(pooled, W, b, bn_gamma, bn_beta, bn_mean, bn_var,
                    ln_gamma, ln_beta)


# speedup vs baseline: 8.9927x; 1.0091x over previous
import functools

import jax
import jax.numpy as jnp
import numpy as np
from jax import lax
from jax.experimental import pallas as pl
from jax.experimental.pallas import tpu as pltpu
from jax.experimental.pallas import tpu_sc as plsc

B, L, V, F = 16384, 20, 100000, 128
NC, NS = 2, 16
NW = NC * NS
ROWS_PER_W = B // NW
IDX_PER_W = ROWS_PER_W * L
SLAB = 128
NSLAB = IDX_PER_W // SLAB

NPHASE = ROWS_PER_W // SLAB
PH_SLABS = NSLAB // NPHASE
_SEG_NP = (
    np.arange(NS, dtype=np.int32)[:, None, None] * (2 * SLAB)
    + np.arange(2, dtype=np.int32)[None, :, None] * SLAB
    + (np.arange(PH_SLABS * SLAB, dtype=np.int32) // L)[None, None, :]
).reshape(NS, 2, PH_SLABS, SLAB)

_EPS = 1e-3
_VEC = 16


_NBUF = 4


def _sc_pool_body(x_r, table, seg, zeros, out, idx_v, seg_v, rows, acc_sh,
                  *sems):
    gsems = sems[:_NBUF]
    dsem = sems[_NBUF:_NBUF + 2]
    zsem = sems[_NBUF + 2:_NBUF + 4]
    cid = lax.axis_index("c")
    sid = lax.axis_index("s")
    wid = sid * NC + cid
    base = sid * (2 * SLAB)

    pltpu.sync_copy(x_r.at[wid], idx_v)
    pltpu.sync_copy(seg.at[sid], seg_v)

    def region(r):
        return acc_sh.at[pl.ds(base + r * SLAB, SLAB)]

    def zero_start(r):
        pltpu.async_copy(zeros, region(r), zsem[r])

    def zero_wait(r):
        pltpu.make_async_copy(zeros, region(r), zsem[r]).wait()

    def drain_start(p, r):
        pltpu.async_copy(region(r),
                         out.at[pl.ds(wid * ROWS_PER_W + p * SLAB, SLAB)],
                         dsem[r])

    def drain_wait(r):
        pltpu.make_async_copy(zeros, region(r), dsem[r]).wait()

    def gather_start(g, b):
        pltpu.async_copy(table.at[idx_v.at[g]], rows.at[b], gsems[b])

    def gather_wait(b):
        pltpu.make_async_copy(table.at[pl.ds(0, SLAB)], rows.at[b],
                              gsems[b]).wait()

    zero_start(0)
    zero_start(1)
    for b in range(_NBUF):
        gather_start(b, b)
    zero_wait(0)

    for p in range(NPHASE):
        r = p % 2
        if p > 0:
            drain_start(p - 1, 1 - r)
            if p == 1:
                zero_wait(1)
            else:
                drain_wait(r)
                zero_start(r)
                zero_wait(r)

        n_loop = PH_SLABS if p < NPHASE - 1 else PH_SLABS - _NBUF

        @pl.loop(0, n_loop, step=_NBUF)
        def _slabs(j):
            for b in range(_NBUF):
                gather_wait(b)
                pltpu.sync_copy(rows.at[b], acc_sh.at[seg_v.at[r, j + b]],
                                add=True)
                gather_start(p * PH_SLABS + j + _NBUF + b, b)

        if p == NPHASE - 1:
            for b in range(_NBUF):
                gather_wait(b)
                pltpu.sync_copy(
                    rows.at[b],
                    acc_sh.at[seg_v.at[r, PH_SLABS - _NBUF + b]], add=True)

    drain_start(NPHASE - 1, 1)
    drain_wait(0)
    drain_wait(1)


_sc_pool = functools.partial(
    pl.kernel,
    out_type=jax.ShapeDtypeStruct((B, F), jnp.float32),
    mesh=plsc.VectorSubcoreMesh(core_axis_name="c", subcore_axis_name="s",
                                num_cores=NC, num_subcores=NS),
    scratch_types=[
        pltpu.VMEM((NSLAB, SLAB), jnp.int32),
        pltpu.VMEM((2, PH_SLABS, SLAB), jnp.int32),
        pltpu.VMEM((_NBUF, SLAB, F), jnp.float32),
        pltpu.VMEM_SHARED((NS * 2 * SLAB, F), jnp.float32),
    ] + [pltpu.SemaphoreType.DMA] * (_NBUF + 4),
)(_sc_pool_body)


def _tc_body(pooled_ref, W_ref, b_ref, bng_ref, bnb_ref, bnm_ref, bnv_ref,
             lng_ref, lnb_ref, out_ref):
    s = bng_ref[...] * lax.rsqrt(bnv_ref[...] + _EPS)
    t = bnb_ref[...] - bnm_ref[...] * s
    z = jnp.dot(pooled_ref[...], W_ref[...],
                preferred_element_type=jnp.float32)
    z = z * (s * (1.0 / L)) + (b_ref[...] * s + t)
    mu = jnp.mean(z, axis=-1, keepdims=True)
    var = jnp.mean(jnp.square(z - mu), axis=-1, keepdims=True)
    out_ref[...] = (z - mu) * lax.rsqrt(var + _EPS) * lng_ref[...] + lnb_ref[...]


_TC_BLK = 1024


def _tc_head(pooled, W, b, bn_gamma, bn_beta, bn_mean, bn_var, ln_gamma, ln_beta):
    vec_spec = pl.BlockSpec((1, F), lambda i: (0, 0))
    return pl.pallas_call(
        _tc_body,
        grid=(B // _TC_BLK,),
        in_specs=[
            pl.BlockSpec((_TC_BLK, F), lambda i: (i, 0)),
            pl.BlockSpec((F, F), lambda i: (0, 0)),
            vec_spec, vec_spec, vec_spec, vec_spec, vec_spec, vec_spec, vec_spec,
        ],
        out_specs=pl.BlockSpec((_TC_BLK, F), lambda i: (i, 0)),
        out_shape=jax.ShapeDtypeStruct((B, F), jnp.float32),
    )(pooled, W, b.reshape(1, F), bn_gamma.reshape(1, F), bn_beta.reshape(1, F),
      bn_mean.reshape(1, F), bn_var.reshape(1, F), ln_gamma.reshape(1, F),
      ln_beta.reshape(1, F))


def kernel(x, table, W, b, bn_gamma, bn_beta, bn_mean, bn_var, ln_gamma, ln_beta):
    x_r = x.astype(jnp.int32).reshape(NW, NSLAB, SLAB)
    seg = jnp.asarray(_SEG_NP)
    zeros = jnp.zeros((SLAB, F), jnp.float32)
    pooled = _sc_pool(x_r, table, seg, zeros)
    return _tc_head(pooled, W, b, bn_gamma, bn_beta, bn_mean, bn_var,
                    ln_gamma, ln_beta)
